# baseline (speedup 1.0000x reference)
import functools

import jax
import jax.numpy as jnp
from jax import lax
from jax.experimental import pallas as pl
from jax.experimental.pallas import tpu as pltpu
from jax.experimental.pallas import tpu_sc as plsc

_NUM_POS = 1500
_DIM = 1024

_NC = 2
_NS = 16
_NW = _NC * _NS

_ROWS_PER_W = 48

_mesh = plsc.VectorSubcoreMesh(core_axis_name="c", subcore_axis_name="s")


@functools.partial(
    pl.kernel,
    mesh=_mesh,
    out_type=jax.ShapeDtypeStruct((_NUM_POS, _DIM), jnp.float32),
    scratch_types=[
        pltpu.VMEM((_ROWS_PER_W, _DIM), jnp.float32),
        pltpu.VMEM((16,), jnp.int32),
        pltpu.SemaphoreType.DMA,
    ],
    compiler_params=pltpu.CompilerParams(skip_device_barrier=True),
)
def _copy_kernel(w_hbm, out_hbm, buf, idx_ref, sem):
    wid = lax.axis_index("s") * _NC + lax.axis_index("c")
    base = wid * _ROWS_PER_W

    @pl.when(wid < _NW - 1)
    def _():
        pltpu.sync_copy(w_hbm.at[pl.ds(base, _ROWS_PER_W), :], buf)
        pltpu.sync_copy(buf, out_hbm.at[pl.ds(base, _ROWS_PER_W), :])

    @pl.when(wid == _NW - 1)
    def _():
        idx_ref[...] = jnp.minimum(
            base + lax.iota(jnp.int32, 16), _NUM_POS - 1)
        pltpu.async_copy(
            w_hbm.at[idx_ref], buf.at[pl.ds(0, 16), :], sem).wait()
        pltpu.async_copy(
            buf.at[pl.ds(0, 16), :], out_hbm.at[idx_ref], sem).wait()


def kernel(input_ids, weight):
    del input_ids
    return _copy_kernel(weight)


# speedup vs baseline: 1.0003x; 1.0003x over previous
import functools

import jax
import jax.numpy as jnp
from jax import lax
from jax.experimental import pallas as pl
from jax.experimental.pallas import tpu as pltpu
from jax.experimental.pallas import tpu_sc as plsc

_NUM_POS = 1500
_DIM = 1024

_NC = 2
_NS = 16
_NW = _NC * _NS

_ROWS_PER_W = 48

_mesh = plsc.VectorSubcoreMesh(core_axis_name="c", subcore_axis_name="s")


@functools.partial(
    pl.kernel,
    mesh=_mesh,
    out_type=jax.ShapeDtypeStruct((_NUM_POS, _DIM), jnp.float32),
    scratch_types=[
        pltpu.VMEM((_ROWS_PER_W, _DIM), jnp.float32),
        pltpu.VMEM((16,), jnp.int32),
        pltpu.SemaphoreType.DMA,
    ],
)
def _copy_kernel(w_hbm, out_hbm, buf, idx_ref, sem):
    wid = lax.axis_index("s") * _NC + lax.axis_index("c")
    base = wid * _ROWS_PER_W

    @pl.when(wid < _NW - 1)
    def _():
        pltpu.sync_copy(w_hbm.at[pl.ds(base, _ROWS_PER_W), :], buf)
        pltpu.sync_copy(buf, out_hbm.at[pl.ds(base, _ROWS_PER_W), :])

    @pl.when(wid == _NW - 1)
    def _():
        idx_ref[...] = jnp.minimum(
            base + lax.iota(jnp.int32, 16), _NUM_POS - 1)
        pltpu.async_copy(
            w_hbm.at[idx_ref], buf.at[pl.ds(0, 16), :], sem).wait()
        pltpu.async_copy(
            buf.at[pl.ds(0, 16), :], out_hbm.at[idx_ref], sem).wait()


def kernel(input_ids, weight):
    del input_ids
    return _copy_kernel(weight)
